# baseline (device time: 41891 ns/iter reference)
import jax
import jax.numpy as jnp
from jax import lax
from jax.experimental import pallas as pl
from jax.experimental.pallas import tpu as pltpu


def kernel(Q, K, V):
    b, sq, h, d = Q.shape
    scale = d ** -0.5

    def body(q_ref, k_ref, v_ref, out_ref, o_buf, ml_buf, send_sems, recv_sems):
        my_x = lax.axis_index("x")
        my_y = lax.axis_index("y")
        peer = (my_x, 1 - my_y)

        barrier = pltpu.get_barrier_semaphore()
        pl.semaphore_signal(
            barrier, inc=1, device_id=peer, device_id_type=pl.DeviceIdType.MESH
        )
        pl.semaphore_wait(barrier, 1)

        q4 = q_ref[...].astype(jnp.bfloat16)
        k4 = k_ref[...].astype(jnp.bfloat16)
        v4 = v_ref[...].astype(jnp.bfloat16)
        s = jnp.sum(k4 * q4, axis=3, dtype=jnp.float32) * scale
        m = jnp.max(s, axis=1, keepdims=True)
        p = jnp.exp(s - m)
        l = jnp.sum(p, axis=1, keepdims=True)
        o = jnp.sum(
            p[..., None].astype(jnp.bfloat16) * v4, axis=1, dtype=jnp.float32
        )
        m_loc = m[:, 0, :]
        l_loc = l[:, 0, :]
        o_loc = o

        o_buf[0] = o_loc
        ml_buf[0, 0] = m_loc
        ml_buf[0, 1] = l_loc

        rdma_o = pltpu.make_async_remote_copy(
            src_ref=o_buf.at[0],
            dst_ref=o_buf.at[1],
            send_sem=send_sems.at[0],
            recv_sem=recv_sems.at[0],
            device_id=peer,
            device_id_type=pl.DeviceIdType.MESH,
        )
        rdma_ml = pltpu.make_async_remote_copy(
            src_ref=ml_buf.at[0],
            dst_ref=ml_buf.at[1],
            send_sem=send_sems.at[1],
            recv_sem=recv_sems.at[1],
            device_id=peer,
            device_id_type=pl.DeviceIdType.MESH,
        )
        rdma_o.start()
        rdma_ml.start()
        rdma_o.wait()
        rdma_ml.wait()

        m_rem = ml_buf[1, 0]
        l_rem = ml_buf[1, 1]
        o_rem = o_buf[1]
        m_tot = jnp.maximum(m_loc, m_rem)
        a_loc = jnp.exp(m_loc - m_tot)
        a_rem = jnp.exp(m_rem - m_tot)
        l_tot = l_loc * a_loc + l_rem * a_rem
        o_tot = o_loc * a_loc[:, :, None] + o_rem * a_rem[:, :, None]
        out_ref[:, 0] = o_tot / l_tot[:, :, None]

    return pl.pallas_call(
        body,
        out_shape=jax.ShapeDtypeStruct((b, sq, h, d), jnp.float32),
        in_specs=[pl.BlockSpec(memory_space=pltpu.VMEM)] * 3,
        out_specs=pl.BlockSpec(memory_space=pltpu.VMEM),
        scratch_shapes=[
            pltpu.VMEM((2, b, h, d), jnp.float32),
            pltpu.VMEM((2, 2, b, h), jnp.float32),
            pltpu.SemaphoreType.DMA((2,)),
            pltpu.SemaphoreType.DMA((2,)),
        ],
        compiler_params=pltpu.CompilerParams(collective_id=0),
    )(Q, K, V)


# device time: 35558 ns/iter; 1.1781x vs baseline; 1.1781x over previous
import jax
import jax.numpy as jnp
from jax import lax
from jax.experimental import pallas as pl
from jax.experimental.pallas import tpu as pltpu


def kernel(Q, K, V):
    b, sq, h, d = Q.shape
    skv = K.shape[1]
    scale = d ** -0.5

    def body(q_ref, k_ref, v_ref, out_ref, o_buf, ml_buf, send_sems, recv_sems):
        my_x = lax.axis_index("x")
        my_y = lax.axis_index("y")
        peer = (my_x, 1 - my_y)

        barrier = pltpu.get_barrier_semaphore()
        pl.semaphore_signal(
            barrier, inc=1, device_id=peer, device_id_type=pl.DeviceIdType.MESH
        )
        pl.semaphore_wait(barrier, 1)

        hd = h * d
        iota = lax.broadcasted_iota
        ETb = (iota(jnp.int32, (h, hd), 1) // d == iota(jnp.int32, (h, hd), 0))
        ETm = ETb.astype(jnp.float32)
        ETb = ETb.astype(jnp.bfloat16)

        k2 = k_ref[...].astype(jnp.bfloat16).reshape(b * skv, hd)
        v2 = v_ref[...].astype(jnp.bfloat16).reshape(b * skv, hd)

        ms, ls, os_ = [], [], []
        for bi in range(b):
            kb2 = k2[bi * skv:(bi + 1) * skv]
            vb2 = v2[bi * skv:(bi + 1) * skv]
            qb = q_ref[bi, 0].astype(jnp.bfloat16)
            qrow = jnp.tile(qb, (1, h)) * ETb
            qblk = qrow.T
            s_kh = lax.dot_general(
                kb2, qblk, (((1,), (0,)), ((), ())),
                preferred_element_type=jnp.float32,
            ) * scale
            m = jnp.max(s_kh, axis=0, keepdims=True)
            p = jnp.exp(s_kh - m)
            l = jnp.sum(p, axis=0, keepdims=True)
            o_t = lax.dot_general(
                p.astype(jnp.bfloat16), vb2, (((0,), (0,)), ((), ())),
                preferred_element_type=jnp.float32,
            )
            o = jnp.sum((o_t * ETm).reshape(h, h, d), axis=1)
            ms.append(m)
            ls.append(l)
            os_.append(o[None])
        m_loc = jnp.concatenate(ms, axis=0)
        l_loc = jnp.concatenate(ls, axis=0)
        o_loc = jnp.concatenate(os_, axis=0)

        o_buf[0] = o_loc
        ml_buf[0, 0] = m_loc
        ml_buf[0, 1] = l_loc

        rdma_o = pltpu.make_async_remote_copy(
            src_ref=o_buf.at[0],
            dst_ref=o_buf.at[1],
            send_sem=send_sems.at[0],
            recv_sem=recv_sems.at[0],
            device_id=peer,
            device_id_type=pl.DeviceIdType.MESH,
        )
        rdma_ml = pltpu.make_async_remote_copy(
            src_ref=ml_buf.at[0],
            dst_ref=ml_buf.at[1],
            send_sem=send_sems.at[1],
            recv_sem=recv_sems.at[1],
            device_id=peer,
            device_id_type=pl.DeviceIdType.MESH,
        )
        rdma_o.start()
        rdma_ml.start()
        rdma_o.wait()
        rdma_ml.wait()

        m_rem = ml_buf[1, 0]
        l_rem = ml_buf[1, 1]
        o_rem = o_buf[1]
        m_tot = jnp.maximum(m_loc, m_rem)
        a_loc = jnp.exp(m_loc - m_tot)
        a_rem = jnp.exp(m_rem - m_tot)
        l_tot = l_loc * a_loc + l_rem * a_rem
        o_tot = o_loc * a_loc[:, :, None] + o_rem * a_rem[:, :, None]
        out_ref[:, 0] = o_tot / l_tot[:, :, None]

    return pl.pallas_call(
        body,
        out_shape=jax.ShapeDtypeStruct((b, sq, h, d), jnp.float32),
        in_specs=[pl.BlockSpec(memory_space=pltpu.VMEM)] * 3,
        out_specs=pl.BlockSpec(memory_space=pltpu.VMEM),
        scratch_shapes=[
            pltpu.VMEM((2, b, h, d), jnp.float32),
            pltpu.VMEM((2, 2, b, h), jnp.float32),
            pltpu.SemaphoreType.DMA((2,)),
            pltpu.SemaphoreType.DMA((2,)),
        ],
        compiler_params=pltpu.CompilerParams(collective_id=0),
    )(Q, K, V)


# device time: 31691 ns/iter; 1.3219x vs baseline; 1.1220x over previous
import jax
import jax.numpy as jnp
from jax import lax
from jax.experimental import pallas as pl
from jax.experimental.pallas import tpu as pltpu


def kernel(Q, K, V):
    b, sq, h, d = Q.shape
    skv = K.shape[1]
    hd = h * d
    bh = b // 2
    scale = d ** -0.5

    my_x = lax.axis_index("x")
    Qh = lax.dynamic_slice_in_dim(Q, my_x * bh, bh, axis=0)
    Kh = lax.dynamic_slice_in_dim(K, my_x * bh, bh, axis=0)
    Vh = lax.dynamic_slice_in_dim(V, my_x * bh, bh, axis=0)

    def body(q_ref, k_ref, v_ref, out_ref, o_buf, ml_buf, fin_buf,
             send_sems, recv_sems):
        my_x = lax.axis_index("x")
        my_y = lax.axis_index("y")
        y_peer = (my_x, 1 - my_y)
        x_peer = (1 - my_x, my_y)

        barrier = pltpu.get_barrier_semaphore()
        for peer in (y_peer, x_peer):
            pl.semaphore_signal(
                barrier, inc=1, device_id=peer,
                device_id_type=pl.DeviceIdType.MESH,
            )
        pl.semaphore_wait(barrier, 2)

        iota = lax.broadcasted_iota
        ETb = (iota(jnp.int32, (h, hd), 1) // d
               == iota(jnp.int32, (h, hd), 0))
        ETm = ETb.astype(jnp.float32)
        ETb = ETb.astype(jnp.bfloat16)

        k2 = k_ref[...].astype(jnp.bfloat16).reshape(bh * skv, hd)
        v2 = v_ref[...].astype(jnp.bfloat16).reshape(bh * skv, hd)

        ms, ls, os_ = [], [], []
        for bi in range(bh):
            kb2 = k2[bi * skv:(bi + 1) * skv]
            vb2 = v2[bi * skv:(bi + 1) * skv]
            qb = q_ref[bi, 0].astype(jnp.bfloat16)
            qrow = jnp.tile(qb, (1, h)) * ETb
            qblk = qrow.T
            s_kh = lax.dot_general(
                kb2, qblk, (((1,), (0,)), ((), ())),
                preferred_element_type=jnp.float32,
            ) * scale
            m = jnp.max(s_kh, axis=0, keepdims=True)
            p = jnp.exp(s_kh - m)
            l = jnp.sum(p, axis=0, keepdims=True)
            o_t = lax.dot_general(
                p.astype(jnp.bfloat16), vb2, (((0,), (0,)), ((), ())),
                preferred_element_type=jnp.float32,
            )
            o = jnp.sum((o_t * ETm).reshape(h, h, d), axis=1)
            ms.append(m)
            ls.append(l)
            os_.append(o[None])
        m_loc = jnp.concatenate(ms, axis=0)
        l_loc = jnp.concatenate(ls, axis=0)
        o_loc = jnp.concatenate(os_, axis=0)

        o_buf[0] = o_loc
        ml_buf[0, 0] = m_loc
        ml_buf[0, 1] = l_loc

        rdma_o = pltpu.make_async_remote_copy(
            src_ref=o_buf.at[0], dst_ref=o_buf.at[1],
            send_sem=send_sems.at[0], recv_sem=recv_sems.at[0],
            device_id=y_peer, device_id_type=pl.DeviceIdType.MESH,
        )
        rdma_ml = pltpu.make_async_remote_copy(
            src_ref=ml_buf.at[0], dst_ref=ml_buf.at[1],
            send_sem=send_sems.at[1], recv_sem=recv_sems.at[1],
            device_id=y_peer, device_id_type=pl.DeviceIdType.MESH,
        )
        rdma_o.start()
        rdma_ml.start()
        rdma_o.wait()
        rdma_ml.wait()

        m_rem = ml_buf[1, 0]
        l_rem = ml_buf[1, 1]
        o_rem = o_buf[1]
        m_tot = jnp.maximum(m_loc, m_rem)
        a_loc = jnp.exp(m_loc - m_tot)
        a_rem = jnp.exp(m_rem - m_tot)
        l_tot = l_loc * a_loc + l_rem * a_rem
        fin = (o_loc * a_loc[:, :, None] + o_rem * a_rem[:, :, None]) \
            / l_tot[:, :, None]
        fin_buf[0] = fin
        out_ref[pl.ds(my_x * bh, bh), 0] = fin

        rdma_fin = pltpu.make_async_remote_copy(
            src_ref=fin_buf.at[0], dst_ref=fin_buf.at[1],
            send_sem=send_sems.at[2], recv_sem=recv_sems.at[2],
            device_id=x_peer, device_id_type=pl.DeviceIdType.MESH,
        )
        rdma_fin.start()
        rdma_fin.wait()
        out_ref[pl.ds((1 - my_x) * bh, bh), 0] = fin_buf[1]

    return pl.pallas_call(
        body,
        out_shape=jax.ShapeDtypeStruct((b, sq, h, d), jnp.float32),
        in_specs=[pl.BlockSpec(memory_space=pltpu.VMEM)] * 3,
        out_specs=pl.BlockSpec(memory_space=pltpu.VMEM),
        scratch_shapes=[
            pltpu.VMEM((2, bh, h, d), jnp.float32),
            pltpu.VMEM((2, 2, bh, h), jnp.float32),
            pltpu.VMEM((2, bh, h, d), jnp.float32),
            pltpu.SemaphoreType.DMA((3,)),
            pltpu.SemaphoreType.DMA((3,)),
        ],
        compiler_params=pltpu.CompilerParams(
            collective_id=0,
            vmem_limit_bytes=100 * 1024 * 1024,
        ),
    )(Qh, Kh, Vh)


# device time: 28669 ns/iter; 1.4612x vs baseline; 1.1054x over previous
import jax
import jax.numpy as jnp
from jax import lax
from jax.experimental import pallas as pl
from jax.experimental.pallas import tpu as pltpu


def kernel(Q, K, V):
    b, sq, h, d = Q.shape
    skv = K.shape[1]
    hd = h * d
    bh = b // 2
    scale = d ** -0.5

    my_x = lax.axis_index("x")
    Qh = lax.dynamic_slice_in_dim(Q, my_x * bh, bh, axis=0)
    Kh = lax.dynamic_slice_in_dim(K, my_x * bh, bh, axis=0).astype(jnp.bfloat16)
    Vh = lax.dynamic_slice_in_dim(V, my_x * bh, bh, axis=0).astype(jnp.bfloat16)

    def body(q_ref, k_ref, v_ref, out_ref, o_buf, ml_buf, fin_buf,
             send_sems, recv_sems):
        my_x = lax.axis_index("x")
        my_y = lax.axis_index("y")
        y_peer = (my_x, 1 - my_y)
        x_peer = (1 - my_x, my_y)

        barrier = pltpu.get_barrier_semaphore()
        for peer in (y_peer, x_peer):
            pl.semaphore_signal(
                barrier, inc=1, device_id=peer,
                device_id_type=pl.DeviceIdType.MESH,
            )
        pl.semaphore_wait(barrier, 2)

        iota = lax.broadcasted_iota
        ETb = (iota(jnp.int32, (h, hd), 1) // d
               == iota(jnp.int32, (h, hd), 0))
        ETm = ETb.astype(jnp.float32)
        ETb = ETb.astype(jnp.bfloat16)

        k2 = k_ref[...].reshape(bh * skv, hd)
        v2 = v_ref[...].reshape(bh * skv, hd)

        ms, ls, os_ = [], [], []
        for bi in range(bh):
            kb2 = k2[bi * skv:(bi + 1) * skv]
            vb2 = v2[bi * skv:(bi + 1) * skv]
            qb = q_ref[bi, 0].astype(jnp.bfloat16)
            qrow = jnp.tile(qb, (1, h)) * ETb
            qblk = qrow.T
            s_kh = lax.dot_general(
                kb2, qblk, (((1,), (0,)), ((), ())),
                preferred_element_type=jnp.float32,
            ) * scale
            m = jnp.max(s_kh, axis=0, keepdims=True)
            p = jnp.exp(s_kh - m)
            l = jnp.sum(p, axis=0, keepdims=True)
            o_t = lax.dot_general(
                p.astype(jnp.bfloat16), vb2, (((0,), (0,)), ((), ())),
                preferred_element_type=jnp.float32,
            )
            o = jnp.sum((o_t * ETm).reshape(h, h, d), axis=1)
            ms.append(m)
            ls.append(l)
            os_.append(o[None])
        m_loc = jnp.concatenate(ms, axis=0)
        l_loc = jnp.concatenate(ls, axis=0)
        o_loc = jnp.concatenate(os_, axis=0)

        o_buf[0] = o_loc
        ml_buf[0, 0] = m_loc
        ml_buf[0, 1] = l_loc

        rdma_o = pltpu.make_async_remote_copy(
            src_ref=o_buf.at[0], dst_ref=o_buf.at[1],
            send_sem=send_sems.at[0], recv_sem=recv_sems.at[0],
            device_id=y_peer, device_id_type=pl.DeviceIdType.MESH,
        )
        rdma_ml = pltpu.make_async_remote_copy(
            src_ref=ml_buf.at[0], dst_ref=ml_buf.at[1],
            send_sem=send_sems.at[1], recv_sem=recv_sems.at[1],
            device_id=y_peer, device_id_type=pl.DeviceIdType.MESH,
        )
        rdma_o.start()
        rdma_ml.start()
        rdma_o.wait()
        rdma_ml.wait()

        m_rem = ml_buf[1, 0]
        l_rem = ml_buf[1, 1]
        o_rem = o_buf[1]
        m_tot = jnp.maximum(m_loc, m_rem)
        a_loc = jnp.exp(m_loc - m_tot)
        a_rem = jnp.exp(m_rem - m_tot)
        l_tot = l_loc * a_loc + l_rem * a_rem
        fin = (o_loc * a_loc[:, :, None] + o_rem * a_rem[:, :, None]) \
            / l_tot[:, :, None]
        fin_buf[0] = fin
        out_ref[pl.ds(my_x * bh, bh), 0] = fin

        rdma_fin = pltpu.make_async_remote_copy(
            src_ref=fin_buf.at[0], dst_ref=fin_buf.at[1],
            send_sem=send_sems.at[2], recv_sem=recv_sems.at[2],
            device_id=x_peer, device_id_type=pl.DeviceIdType.MESH,
        )
        rdma_fin.start()
        rdma_fin.wait()
        out_ref[pl.ds((1 - my_x) * bh, bh), 0] = fin_buf[1]

    return pl.pallas_call(
        body,
        out_shape=jax.ShapeDtypeStruct((b, sq, h, d), jnp.float32),
        in_specs=[pl.BlockSpec(memory_space=pltpu.VMEM)] * 3,
        out_specs=pl.BlockSpec(memory_space=pltpu.VMEM),
        scratch_shapes=[
            pltpu.VMEM((2, bh, h, d), jnp.float32),
            pltpu.VMEM((2, 2, bh, h), jnp.float32),
            pltpu.VMEM((2, bh, h, d), jnp.float32),
            pltpu.SemaphoreType.DMA((3,)),
            pltpu.SemaphoreType.DMA((3,)),
        ],
        compiler_params=pltpu.CompilerParams(
            collective_id=0,
            vmem_limit_bytes=100 * 1024 * 1024,
        ),
    )(Qh, Kh, Vh)


# device time: 27582 ns/iter; 1.5188x vs baseline; 1.0394x over previous
import jax
import jax.numpy as jnp
from jax import lax
from jax.experimental import pallas as pl
from jax.experimental.pallas import tpu as pltpu


def kernel(Q, K, V):
    b, sq, h, d = Q.shape
    skv = K.shape[1]
    hd = h * d
    bh = b // 2
    scale = d ** -0.5

    my_x = lax.axis_index("x")
    Qh = lax.dynamic_slice_in_dim(Q, my_x * bh, bh, axis=0)
    Kh = lax.dynamic_slice_in_dim(K, my_x * bh, bh, axis=0).astype(jnp.bfloat16)
    Vh = lax.dynamic_slice_in_dim(V, my_x * bh, bh, axis=0).astype(jnp.bfloat16)

    def body(q_ref, k_ref, v_ref, out_ref, o_buf, ml_buf,
             send_sems, recv_sems):
        my_x = lax.axis_index("x")
        my_y = lax.axis_index("y")
        y_peer = (my_x, 1 - my_y)
        x_peer = (1 - my_x, my_y)
        dg_peer = (1 - my_x, 1 - my_y)
        peers = (y_peer, x_peer, dg_peer)

        barrier = pltpu.get_barrier_semaphore()
        for peer in peers:
            pl.semaphore_signal(
                barrier, inc=1, device_id=peer,
                device_id_type=pl.DeviceIdType.MESH,
            )
        pl.semaphore_wait(barrier, 3)

        iota = lax.broadcasted_iota
        ETb = (iota(jnp.int32, (h, hd), 1) // d
               == iota(jnp.int32, (h, hd), 0))
        ETm = ETb.astype(jnp.float32)
        ETb = ETb.astype(jnp.bfloat16)

        k2 = k_ref[...].reshape(bh * skv, hd)
        v2 = v_ref[...].reshape(bh * skv, hd)

        ms, ls, os_ = [], [], []
        for bi in range(bh):
            kb2 = k2[bi * skv:(bi + 1) * skv]
            vb2 = v2[bi * skv:(bi + 1) * skv]
            qb = q_ref[bi, 0].astype(jnp.bfloat16)
            qrow = jnp.tile(qb, (1, h)) * ETb
            qblk = qrow.T
            s_kh = lax.dot_general(
                kb2, qblk, (((1,), (0,)), ((), ())),
                preferred_element_type=jnp.float32,
            ) * scale
            m = jnp.max(s_kh, axis=0, keepdims=True)
            p = jnp.exp(s_kh - m)
            l = jnp.sum(p, axis=0, keepdims=True)
            o_t = lax.dot_general(
                p.astype(jnp.bfloat16), vb2, (((0,), (0,)), ((), ())),
                preferred_element_type=jnp.float32,
            )
            o = jnp.sum((o_t * ETm).reshape(h, h, d), axis=1)
            ms.append(m)
            ls.append(l)
            os_.append(o[None])
        m_loc = jnp.concatenate(ms, axis=0)
        l_loc = jnp.concatenate(ls, axis=0)
        o_loc = jnp.concatenate(os_, axis=0)

        o_buf[0] = o_loc
        ml_buf[0, 0] = m_loc
        ml_buf[0, 1] = l_loc

        rdmas = []
        for idx, peer in enumerate(peers):
            slot = idx + 1
            rdmas.append(pltpu.make_async_remote_copy(
                src_ref=o_buf.at[0], dst_ref=o_buf.at[slot],
                send_sem=send_sems.at[idx], recv_sem=recv_sems.at[idx],
                device_id=peer, device_id_type=pl.DeviceIdType.MESH,
            ))
            rdmas.append(pltpu.make_async_remote_copy(
                src_ref=ml_buf.at[0], dst_ref=ml_buf.at[slot],
                send_sem=send_sems.at[3 + idx], recv_sem=recv_sems.at[3 + idx],
                device_id=peer, device_id_type=pl.DeviceIdType.MESH,
            ))
        for r in rdmas:
            r.start()
        for r in rdmas:
            r.wait()

        def merge(sa, sb):
            m_a, l_a, o_a = ml_buf[sa, 0], ml_buf[sa, 1], o_buf[sa]
            m_b, l_b, o_b = ml_buf[sb, 0], ml_buf[sb, 1], o_buf[sb]
            m_t = jnp.maximum(m_a, m_b)
            a_a = jnp.exp(m_a - m_t)
            a_b = jnp.exp(m_b - m_t)
            l_t = l_a * a_a + l_b * a_b
            return (o_a * a_a[:, :, None] + o_b * a_b[:, :, None]) \
                / l_t[:, :, None]

        out_ref[pl.ds(my_x * bh, bh), 0] = merge(0, 1)
        out_ref[pl.ds((1 - my_x) * bh, bh), 0] = merge(2, 3)

    return pl.pallas_call(
        body,
        out_shape=jax.ShapeDtypeStruct((b, sq, h, d), jnp.float32),
        in_specs=[pl.BlockSpec(memory_space=pltpu.VMEM)] * 3,
        out_specs=pl.BlockSpec(memory_space=pltpu.VMEM),
        scratch_shapes=[
            pltpu.VMEM((4, bh, h, d), jnp.float32),
            pltpu.VMEM((4, 2, bh, h), jnp.float32),
            pltpu.SemaphoreType.DMA((6,)),
            pltpu.SemaphoreType.DMA((6,)),
        ],
        compiler_params=pltpu.CompilerParams(
            collective_id=0,
            vmem_limit_bytes=100 * 1024 * 1024,
        ),
    )(Qh, Kh, Vh)
